# baseline (speedup 1.0000x reference)
import functools

import jax
import jax.numpy as jnp
from jax.experimental import pallas as pl
from jax.experimental.pallas import tpu as pltpu

K = 32


def _toeplitz64(w):
    jj = jnp.arange(K)[:, None]
    ss = jnp.arange(2 * K)[None, :]
    d = ss - jj
    wf = w.astype(jnp.float32)
    return jnp.where((d >= 0) & (d < K), wf[jnp.clip(d, 0, K - 1)], 0.0)


def _conv_pool_rows(x3, halo, w64):
    batch, tp = x3.shape[0], x3.shape[1]
    rows = []
    for bb in range(batch):
        a = x3[bb].T
        hcol = halo[bb:bb + 1, :].T
        ahi = jnp.concatenate([a[:, 1:], hcol], axis=1)
        s = jnp.concatenate([a, ahi], axis=0)
        y = jnp.dot(w64, s, preferred_element_type=jnp.float32)
        rows.append(jnp.max(y, axis=0, keepdims=True))
    return jnp.concatenate(rows, axis=0)


def _stage1_kernel(x_ref, h_ref, w_ref, b_ref, o_ref, *, tp, n_valid, batch):
    i = pl.program_id(0)
    x3 = x_ref[...].reshape(batch, tp, K)
    halo = h_ref[...].reshape(batch, 128)[:, :K]
    pooled = _conv_pool_rows(x3, halo, w_ref[...]) + b_ref[0]
    pooled = jnp.maximum(pooled, 0.0)
    col = i * tp + jax.lax.broadcasted_iota(jnp.int32, (1, tp), 1)
    o_ref[...] = jnp.where(col < n_valid, pooled, 0.0)


def _stage1(x3d, w, b, *, tp, out_cols):
    B, _, L = x3d.shape
    P = (L - K + 1) // K
    ntiles = pl.cdiv(max(P, out_cols), tp)
    tpl = tp * K
    last_h = L // 128 - 1

    body = functools.partial(_stage1_kernel, tp=tp, n_valid=P, batch=B)
    return pl.pallas_call(
        body,
        out_shape=jax.ShapeDtypeStruct((B, out_cols), jnp.float32),
        grid=(ntiles,),
        in_specs=[
            pl.BlockSpec((B, 1, tpl), lambda i: (0, 0, i)),
            pl.BlockSpec((B, 1, 128),
                         lambda i: (0, 0, jnp.minimum((i + 1) * (tpl // 128),
                                                      last_h))),
            pl.BlockSpec((K, 2 * K), lambda i: (0, 0)),
            pl.BlockSpec(memory_space=pltpu.MemorySpace.SMEM),
        ],
        out_specs=pl.BlockSpec((B, tp), lambda i: (0, i)),
        compiler_params=pltpu.CompilerParams(
            dimension_semantics=("parallel",),
            vmem_limit_bytes=48 * 1024 * 1024,
        ),
    )(x3d, x3d, _toeplitz64(w), jnp.reshape(b, (1,)).astype(jnp.float32))


def _tail_kernel(h_ref, w64_ref, cb_ref, w1_ref, b1_ref, w2_ref, b2_ref,
                 o_ref, *, batch, n2, p2):
    h3 = h_ref[...].reshape(batch, n2, K)
    halo = h3[:, n2 - 1, :]
    h2 = _conv_pool_rows(h3, halo, w64_ref[...]) + cb_ref[0]
    h2 = jnp.maximum(h2, 0.0)[:, :p2]

    dn = (((1,), (1,)), ((), ()))
    a1 = jax.lax.dot_general(h2, w1_ref[...], dn,
                             preferred_element_type=jnp.float32) + b1_ref[...]
    a1 = jnp.maximum(a1, 0.0)
    z = jax.lax.dot_general(a1, w2_ref[...], dn,
                            preferred_element_type=jnp.float32) + b2_ref[...]
    z = z - jnp.max(z, axis=-1, keepdims=True)
    e = jnp.exp(z)
    o_ref[...] = e / jnp.sum(e, axis=-1, keepdims=True)


def _tail(h1, c2w, c2b, w1, b1, w2, b2, *, p2):
    B, oc = h1.shape
    n2 = oc // K
    n_cls = w2.shape[0]
    body = functools.partial(_tail_kernel, batch=B, n2=n2, p2=p2)
    vmem = pl.BlockSpec(memory_space=pltpu.MemorySpace.VMEM)
    return pl.pallas_call(
        body,
        out_shape=jax.ShapeDtypeStruct((B, n_cls), jnp.float32),
        in_specs=[vmem, vmem,
                  pl.BlockSpec(memory_space=pltpu.MemorySpace.SMEM),
                  vmem, vmem, vmem, vmem],
        out_specs=vmem,
        compiler_params=pltpu.CompilerParams(
            vmem_limit_bytes=32 * 1024 * 1024,
        ),
    )(h1, _toeplitz64(c2w), jnp.reshape(c2b, (1,)).astype(jnp.float32),
      w1, b1.reshape(1, -1), w2, b2.reshape(1, -1))


def kernel(conv1_w, conv1_b, conv2_w, conv2_b, lin1_w, lin1_b, lin2_w, lin2_b, x):
    x3d = x.astype(jnp.float32)
    L = x3d.shape[2]
    p1 = (L - K + 1) // K
    p2 = (p1 - K + 1) // K
    tp = 2048
    out_cols = pl.cdiv(p1, tp) * tp
    h1 = _stage1(x3d, conv1_w, conv1_b, tp=tp, out_cols=out_cols)
    out = _tail(h1, conv2_w, conv2_b, lin1_w, lin1_b, lin2_w, lin2_b, p2=p2)
    order = jnp.argsort(-out, axis=0)
    pred = jnp.sort(order[:, 0:2], axis=-1)
    return out, pred


# speedup vs baseline: 1.3635x; 1.3635x over previous
import functools

import jax
import jax.numpy as jnp
from jax.experimental import pallas as pl
from jax.experimental.pallas import tpu as pltpu

K = 32


def _toeplitz64(w):
    jj = jnp.arange(K)[:, None]
    ss = jnp.arange(2 * K)[None, :]
    d = ss - jj
    wf = w.astype(jnp.float32)
    return jnp.where((d >= 0) & (d < K), wf[jnp.clip(d, 0, K - 1)], 0.0)


def _conv_pool_rows(x3, halo, w64):
    batch, tp = x3.shape[0], x3.shape[1]
    rows = []
    for bb in range(batch):
        a = x3[bb].T
        hcol = halo[bb:bb + 1, :].T
        ahi = jnp.concatenate([a[:, 1:], hcol], axis=1)
        s = jnp.concatenate([a, ahi], axis=0)
        y = jnp.dot(w64, s, preferred_element_type=jnp.float32)
        rows.append(jnp.max(y, axis=0, keepdims=True))
    return jnp.concatenate(rows, axis=0)


def _stage1_kernel(x_ref, h_ref, w_ref, b_ref, o_ref, *, tp, n_valid, batch):
    i = pl.program_id(0)
    x3 = x_ref[...].reshape(batch, tp, K)
    halo = h_ref[:, :K]
    pooled = _conv_pool_rows(x3, halo, w_ref[...]) + b_ref[0]
    pooled = jnp.maximum(pooled, 0.0)
    col = i * tp + jax.lax.broadcasted_iota(jnp.int32, (1, tp), 1)
    o_ref[...] = jnp.where(col < n_valid, pooled, 0.0)


def _stage1(xs, w, b, *, tp, out_cols):
    B, L = xs.shape
    P = (L - K + 1) // K
    ntiles = pl.cdiv(max(P, out_cols), tp)
    tpl = tp * K
    last_h = L // 128 - 1

    body = functools.partial(_stage1_kernel, tp=tp, n_valid=P, batch=B)
    return pl.pallas_call(
        body,
        out_shape=jax.ShapeDtypeStruct((B, out_cols), jnp.float32),
        grid=(ntiles,),
        in_specs=[
            pl.BlockSpec((B, tpl), lambda i: (0, i)),
            pl.BlockSpec((B, 128),
                         lambda i: (0, jnp.minimum((i + 1) * (tpl // 128),
                                                   last_h))),
            pl.BlockSpec((K, 2 * K), lambda i: (0, 0)),
            pl.BlockSpec(memory_space=pltpu.MemorySpace.SMEM),
        ],
        out_specs=pl.BlockSpec((B, tp), lambda i: (0, i)),
        compiler_params=pltpu.CompilerParams(
            dimension_semantics=("parallel",),
            vmem_limit_bytes=48 * 1024 * 1024,
        ),
    )(xs, xs, _toeplitz64(w), jnp.reshape(b, (1,)).astype(jnp.float32))


def _tail_kernel(h_ref, w64_ref, cb_ref, w1_ref, b1_ref, w2_ref, b2_ref,
                 o_ref, *, batch, n2, p2):
    h3 = h_ref[...].reshape(batch, n2, K)
    halo = h3[:, n2 - 1, :]
    h2 = _conv_pool_rows(h3, halo, w64_ref[...]) + cb_ref[0]
    h2 = jnp.maximum(h2, 0.0)[:, :p2]

    dn = (((1,), (1,)), ((), ()))
    a1 = jax.lax.dot_general(h2, w1_ref[...], dn,
                             preferred_element_type=jnp.float32) + b1_ref[...]
    a1 = jnp.maximum(a1, 0.0)
    z = jax.lax.dot_general(a1, w2_ref[...], dn,
                            preferred_element_type=jnp.float32) + b2_ref[...]
    z = z - jnp.max(z, axis=-1, keepdims=True)
    e = jnp.exp(z)
    o_ref[...] = e / jnp.sum(e, axis=-1, keepdims=True)


def _tail(h1, c2w, c2b, w1, b1, w2, b2, *, p2):
    B, oc = h1.shape
    n2 = oc // K
    n_cls = w2.shape[0]
    body = functools.partial(_tail_kernel, batch=B, n2=n2, p2=p2)
    vmem = pl.BlockSpec(memory_space=pltpu.MemorySpace.VMEM)
    return pl.pallas_call(
        body,
        out_shape=jax.ShapeDtypeStruct((B, n_cls), jnp.float32),
        in_specs=[vmem, vmem,
                  pl.BlockSpec(memory_space=pltpu.MemorySpace.SMEM),
                  vmem, vmem, vmem, vmem],
        out_specs=vmem,
        compiler_params=pltpu.CompilerParams(
            vmem_limit_bytes=32 * 1024 * 1024,
        ),
    )(h1, _toeplitz64(c2w), jnp.reshape(c2b, (1,)).astype(jnp.float32),
      w1, b1.reshape(1, -1), w2, b2.reshape(1, -1))


def kernel(conv1_w, conv1_b, conv2_w, conv2_b, lin1_w, lin1_b, lin2_w, lin2_b, x):
    xs = jnp.reshape(x.astype(jnp.float32), (x.shape[0], x.shape[2]))
    L = xs.shape[1]
    p1 = (L - K + 1) // K
    p2 = (p1 - K + 1) // K
    tp = 2048
    out_cols = pl.cdiv(p1, tp) * tp
    h1 = _stage1(xs, conv1_w, conv1_b, tp=tp, out_cols=out_cols)
    out = _tail(h1, conv2_w, conv2_b, lin1_w, lin1_b, lin2_w, lin2_b, p2=p2)
    order = jnp.argsort(-out, axis=0)
    pred = jnp.sort(order[:, 0:2], axis=-1)
    return out, pred
